# Initial kernel scaffold; baseline (speedup 1.0000x reference)
import jax
import jax.numpy as jnp
from jax.experimental import pallas as pl


def kernel(centers, contexts, neg_contexts, item_embed, context_embed):
    raise NotImplementedError("write your pallas kernel here")


# speedup vs baseline: 5.2876x; 5.2876x over previous
import functools

import jax
import jax.numpy as jnp
from jax import lax
from jax.experimental import pallas as pl
from jax.experimental.pallas import tpu as pltpu
from jax.experimental.pallas import tpu_sc as plsc

_D = 64
_B = 16384
_K = 20
_KP1 = _K + 1
_KPAD = 32

_NW = 32
_PER_W = _B // _NW
_C = 64
_NCHUNK = _PER_W // _C
_R = _C * _KP1
_GW = 84
_NG = _R // _GW


def _sc_logits(centers, all_idx2d, item_tab, ctx_tab):
    mesh = plsc.VectorSubcoreMesh(core_axis_name="c", subcore_axis_name="s")

    @functools.partial(
        pl.kernel,
        mesh=mesh,
        compiler_params=pltpu.CompilerParams(use_tc_tiling_on_sc=False),
        out_type=jax.ShapeDtypeStruct((_B, _KPAD), jnp.float32),
        scratch_types=[
            pltpu.VMEM((_C,), jnp.int32),
            pltpu.VMEM((_NG, _GW), jnp.int32),
            pltpu.VMEM((_C, _D), jnp.float32),
            pltpu.VMEM((_R, _D), jnp.float32),
            pltpu.VMEM((_C, _KPAD), jnp.float32),
            pltpu.SemaphoreType.DMA,
        ],
    )
    def sc_kernel(cen_hbm, aidx_hbm, item_hbm, ctx_hbm, out_hbm,
                  cen_idx_v, aidx_v, cen_rows_v, rows_v, logit_v, sem):
        wid = lax.axis_index("s") * 2 + lax.axis_index("c")
        base_b = wid * _PER_W

        lane = lax.iota(jnp.int32, 16)
        lane_eq = [lane == i for i in range(16)]
        perms = [lane ^ sh for sh in (8, 4, 2, 1)]
        dnums = lax.GatherDimensionNumbers(
            offset_dims=(), collapsed_slice_dims=(0,), start_index_map=(0,))

        def lane_sum(v):
            for perm in perms:
                v = v + lax.gather(
                    v, perm[:, None], dimension_numbers=dnums,
                    slice_sizes=(1,),
                    mode=lax.GatherScatterMode.PROMISE_IN_BOUNDS)
            return v

        for g in range(_NCHUNK):
            b0 = pl.multiple_of(base_b + g * _C, _C)
            pltpu.sync_copy(cen_hbm.at[pl.ds(b0, _C)], cen_idx_v)
            pltpu.sync_copy(
                aidx_hbm.at[pl.ds(pl.multiple_of(b0 * _KP1 // _GW, _NG), _NG),
                            :], aidx_v)
            cps = [pltpu.async_copy(item_hbm.at[cen_idx_v], cen_rows_v, sem)]
            for j in range(_NG):
                cps.append(pltpu.async_copy(
                    ctx_hbm.at[aidx_v.at[j]],
                    rows_v.at[pl.ds(j * _GW, _GW), :], sem))
            for cp in cps:
                cp.wait()

            def body(b, carry):
                c0 = cen_rows_v[b, pl.ds(0, 16)]
                c1 = cen_rows_v[b, pl.ds(16, 16)]
                c2 = cen_rows_v[b, pl.ds(32, 16)]
                c3 = cen_rows_v[b, pl.ds(48, 16)]
                rbase = b * _KP1
                acc0 = jnp.zeros((16,), jnp.float32)
                acc1 = jnp.zeros((16,), jnp.float32)
                for kk in range(_KP1):
                    r = rbase + kk
                    p = (rows_v[r, pl.ds(0, 16)] * c0
                         + rows_v[r, pl.ds(16, 16)] * c1
                         + rows_v[r, pl.ds(32, 16)] * c2
                         + rows_v[r, pl.ds(48, 16)] * c3)
                    s = lane_sum(p)
                    if kk == 0:
                        acc0 = jnp.where(lane_eq[0], -s, acc0)
                    elif kk < 16:
                        acc0 = jnp.where(lane_eq[kk], s, acc0)
                    else:
                        acc1 = jnp.where(lane_eq[kk - 16], s, acc1)
                logit_v[b, pl.ds(0, 16)] = acc0
                logit_v[b, pl.ds(16, 16)] = acc1
                return carry

            lax.fori_loop(0, _C, body, None)
            pltpu.sync_copy(logit_v, out_hbm.at[pl.ds(b0, _C), :])

    return sc_kernel(centers, all_idx2d, item_tab, ctx_tab)


def _tc_loss(logits2d):
    def body(x_ref, o_ref):
        x = x_ref[...]
        col = lax.broadcasted_iota(jnp.int32, x.shape, 1) % _KPAD
        sp = jnp.where(col < _KP1, jnp.log1p(jnp.exp(x)), 0.0)
        o_ref[...] = (jnp.sum(sp) * (1.0 / _B)).reshape(1, 1)

    return pl.pallas_call(
        body,
        out_shape=jax.ShapeDtypeStruct((1, 1), jnp.float32),
    )(logits2d)


def kernel(centers, contexts, neg_contexts, item_embed, context_embed):
    centers = centers.astype(jnp.int32)
    all_idx = jnp.concatenate(
        [contexts[:, None], neg_contexts], axis=1).astype(jnp.int32)
    all_idx2d = all_idx.reshape(_B * _KP1 // _GW, _GW)
    logits = _sc_logits(centers, all_idx2d, item_embed, context_embed)
    loss = _tc_loss(logits.reshape(_B * _KPAD // 128, 128))
    return loss[0, 0]


# speedup vs baseline: 6.6013x; 1.2485x over previous
import functools

import jax
import jax.numpy as jnp
from jax import lax
from jax.experimental import pallas as pl
from jax.experimental.pallas import tpu as pltpu
from jax.experimental.pallas import tpu_sc as plsc

_D = 64
_B = 16384
_K = 20
_KP1 = _K + 1
_KPAD = 32
_V = 1000000

_NW = 32
_PER_W = _B // _NW
_C = 32
_NCHUNK = _PER_W // _C
_R = _C * _KP1
_GW = 96
_NG = _R // _GW

_TBLK = 4096
_TGRID = -(-_V // _TBLK)

_MESH = dict(core_axis_name="c", subcore_axis_name="s")


def _tc_widen(tab_t):
    def body(x_ref, o_ref):
        xt = jnp.swapaxes(x_ref[...], 0, 1)
        o_ref[...] = jnp.concatenate([xt, xt], axis=1)

    return pl.pallas_call(
        body,
        grid=(_TGRID,),
        in_specs=[pl.BlockSpec((_D, _TBLK), lambda i: (0, i))],
        out_specs=pl.BlockSpec((_TBLK, 128), lambda i: (i, 0)),
        out_shape=jax.ShapeDtypeStruct((_V, 128), jnp.float32),
    )(tab_t)


def _sc_logits(centers, all_idx, item2, ctx2):
    mesh = plsc.VectorSubcoreMesh(**_MESH)

    @functools.partial(
        pl.kernel,
        mesh=mesh,
        compiler_params=pltpu.CompilerParams(use_tc_tiling_on_sc=True),
        out_type=jax.ShapeDtypeStruct((_B, _KPAD), jnp.float32),
        scratch_types=[
            pltpu.VMEM((_C,), jnp.int32),
            pltpu.VMEM((_R,), jnp.int32),
            pltpu.VMEM((_C, 128), jnp.float32),
            pltpu.VMEM((_R, 128), jnp.float32),
            pltpu.VMEM((_C, _KPAD), jnp.float32),
            pltpu.SemaphoreType.DMA,
        ],
    )
    def sc_g(cen_hbm, aidx_hbm, item2_hbm, ctx2_hbm, out_hbm,
             cen_idx_v, aidx_v, cen_rows_v, rows_v, logit_v, sem):
        wid = lax.axis_index("s") * 2 + lax.axis_index("c")
        base_b = wid * _PER_W

        lane = lax.iota(jnp.int32, 16)
        lane_eq = [lane == i for i in range(16)]
        perms = [lane ^ sh for sh in (8, 4, 2, 1)]
        dnums = lax.GatherDimensionNumbers(
            offset_dims=(), collapsed_slice_dims=(0,), start_index_map=(0,))

        def lane_sum(v):
            for perm in perms:
                v = v + lax.gather(
                    v, perm[:, None], dimension_numbers=dnums,
                    slice_sizes=(1,),
                    mode=lax.GatherScatterMode.PROMISE_IN_BOUNDS)
            return v

        for g in range(_NCHUNK):
            b0 = pl.multiple_of(base_b + g * _C, _C)
            pltpu.sync_copy(cen_hbm.at[pl.ds(b0, _C)], cen_idx_v)
            pltpu.sync_copy(aidx_hbm.at[pl.ds(b0 * _KP1, _R)], aidx_v)
            cps = [pltpu.async_copy(item2_hbm.at[cen_idx_v],
                                    cen_rows_v, sem)]
            for j in range(_NG):
                cps.append(pltpu.async_copy(
                    ctx2_hbm.at[aidx_v.at[pl.ds(j * _GW, _GW)]],
                    rows_v.at[pl.ds(j * _GW, _GW), :], sem))
            for cp in cps:
                cp.wait()

            def body(b, carry):
                c0 = cen_rows_v[b, pl.ds(0, 16)]
                c1 = cen_rows_v[b, pl.ds(16, 16)]
                c2 = cen_rows_v[b, pl.ds(32, 16)]
                c3 = cen_rows_v[b, pl.ds(48, 16)]
                rbase = b * _KP1
                acc0 = jnp.zeros((16,), jnp.float32)
                acc1 = jnp.zeros((16,), jnp.float32)
                for kk in range(_KP1):
                    r = rbase + kk
                    p = (rows_v[r, pl.ds(0, 16)] * c0
                         + rows_v[r, pl.ds(16, 16)] * c1
                         + rows_v[r, pl.ds(32, 16)] * c2
                         + rows_v[r, pl.ds(48, 16)] * c3)
                    s = lane_sum(p)
                    if kk == 0:
                        acc0 = jnp.where(lane_eq[0], -s, acc0)
                    elif kk < 16:
                        acc0 = jnp.where(lane_eq[kk], s, acc0)
                    else:
                        acc1 = jnp.where(lane_eq[kk - 16], s, acc1)
                logit_v[b, pl.ds(0, 16)] = acc0
                logit_v[b, pl.ds(16, 16)] = acc1
                return carry

            lax.fori_loop(0, _C, body, None)
            pltpu.sync_copy(logit_v, out_hbm.at[pl.ds(b0, _C), :])

    return sc_g(centers, all_idx, item2, ctx2)


def _tc_loss(logits):
    def body(x_ref, o_ref):
        x = x_ref[...]
        col = lax.broadcasted_iota(jnp.int32, x.shape, 1)
        sp = jnp.where(col < _KP1, jnp.log1p(jnp.exp(x)), 0.0)
        o_ref[...] = (jnp.sum(sp) * (1.0 / _B)).reshape(1, 1)

    return pl.pallas_call(
        body,
        out_shape=jax.ShapeDtypeStruct((1, 1), jnp.float32),
    )(logits)


def kernel(centers, contexts, neg_contexts, item_embed, context_embed):
    centers = centers.astype(jnp.int32)
    all_idx = jnp.concatenate(
        [contexts[:, None], neg_contexts], axis=1).astype(jnp.int32)
    all_idx = all_idx.reshape(_B * _KP1)
    item2 = _tc_widen(item_embed.T)
    ctx2 = _tc_widen(context_embed.T)
    logits = _sc_logits(centers, all_idx, item2, ctx2)
    return _tc_loss(logits)[0, 0]


# speedup vs baseline: 6.9850x; 1.0581x over previous
import functools

import jax
import jax.numpy as jnp
from jax import lax
from jax.experimental import pallas as pl
from jax.experimental.pallas import tpu as pltpu
from jax.experimental.pallas import tpu_sc as plsc

_D = 64
_B = 16384
_K = 20
_KP1 = _K + 1
_KPAD = 32
_V = 1000000

_NW = 32
_PER_W = _B // _NW
_C = 64
_NCHUNK = _PER_W // _C
_R = _C * _KP1
_GW = 84
_NG = _R // _GW

_TBLK = 4096
_TGRID = -(-_V // _TBLK)

_MESH = dict(core_axis_name="c", subcore_axis_name="s")


def _tc_widen(tab_t):
    def body(x_ref, o_ref):
        xt = jnp.swapaxes(x_ref[...], 0, 1)
        o_ref[...] = jnp.concatenate([xt, xt], axis=1)

    return pl.pallas_call(
        body,
        grid=(_TGRID,),
        in_specs=[pl.BlockSpec((_D, _TBLK), lambda i: (0, i))],
        out_specs=pl.BlockSpec((_TBLK, 128), lambda i: (i, 0)),
        out_shape=jax.ShapeDtypeStruct((_V, 128), jnp.float32),
    )(tab_t)


def _sc_logits(centers, all_idx2d, item2, ctx2):
    mesh = plsc.VectorSubcoreMesh(**_MESH)

    @functools.partial(
        pl.kernel,
        mesh=mesh,
        compiler_params=pltpu.CompilerParams(use_tc_tiling_on_sc=False),
        out_type=jax.ShapeDtypeStruct((_B, _KPAD), jnp.float32),
        scratch_types=[
            pltpu.VMEM((_C,), jnp.int32),
            pltpu.VMEM((_NG, _GW), jnp.int32),
            pltpu.VMEM((_C, _D), jnp.float32),
            pltpu.VMEM((_R, _D), jnp.float32),
            pltpu.VMEM((_C, _KPAD), jnp.float32),
            pltpu.SemaphoreType.DMA,
        ],
    )
    def sc_g(cen_hbm, aidx_hbm, item2_hbm, ctx2_hbm, out_hbm,
             cen_idx_v, aidx_v, cen_rows_v, rows_v, logit_v, sem):
        wid = lax.axis_index("s") * 2 + lax.axis_index("c")
        base_b = wid * _PER_W

        lane = lax.iota(jnp.int32, 16)
        lane_eq = [lane == i for i in range(16)]
        perms = [lane ^ sh for sh in (8, 4, 2, 1)]
        dnums = lax.GatherDimensionNumbers(
            offset_dims=(), collapsed_slice_dims=(0,), start_index_map=(0,))

        def lane_sum(v):
            for perm in perms:
                v = v + lax.gather(
                    v, perm[:, None], dimension_numbers=dnums,
                    slice_sizes=(1,),
                    mode=lax.GatherScatterMode.PROMISE_IN_BOUNDS)
            return v

        for g in range(_NCHUNK):
            b0 = pl.multiple_of(base_b + g * _C, _C)
            pltpu.sync_copy(cen_hbm.at[pl.ds(b0, _C)], cen_idx_v)
            pltpu.sync_copy(
                aidx_hbm.at[pl.ds(pl.multiple_of(b0 * _KP1 // _GW, _NG), _NG),
                            :], aidx_v)
            cps = [pltpu.async_copy(item2_hbm.at[cen_idx_v],
                                    cen_rows_v, sem)]
            for j in range(_NG):
                cps.append(pltpu.async_copy(
                    ctx2_hbm.at[aidx_v.at[j]],
                    rows_v.at[pl.ds(j * _GW, _GW), :], sem))
            for cp in cps:
                cp.wait()

            def body(b, carry):
                c0 = cen_rows_v[b, pl.ds(0, 16)]
                c1 = cen_rows_v[b, pl.ds(16, 16)]
                c2 = cen_rows_v[b, pl.ds(32, 16)]
                c3 = cen_rows_v[b, pl.ds(48, 16)]
                rbase = b * _KP1
                acc0 = jnp.zeros((16,), jnp.float32)
                acc1 = jnp.zeros((16,), jnp.float32)
                for kk in range(_KP1):
                    r = rbase + kk
                    p = (rows_v[r, pl.ds(0, 16)] * c0
                         + rows_v[r, pl.ds(16, 16)] * c1
                         + rows_v[r, pl.ds(32, 16)] * c2
                         + rows_v[r, pl.ds(48, 16)] * c3)
                    s = lane_sum(p)
                    if kk == 0:
                        acc0 = jnp.where(lane_eq[0], -s, acc0)
                    elif kk < 16:
                        acc0 = jnp.where(lane_eq[kk], s, acc0)
                    else:
                        acc1 = jnp.where(lane_eq[kk - 16], s, acc1)
                logit_v[b, pl.ds(0, 16)] = acc0
                logit_v[b, pl.ds(16, 16)] = acc1
                return carry

            lax.fori_loop(0, _C, body, None)
            pltpu.sync_copy(logit_v, out_hbm.at[pl.ds(b0, _C), :])

    return sc_g(centers, all_idx2d, item2, ctx2)


def _tc_loss(logits):
    def body(x_ref, o_ref):
        x = x_ref[...]
        col = lax.broadcasted_iota(jnp.int32, x.shape, 1) % _KPAD
        sp = jnp.where(col < _KP1, jnp.log1p(jnp.exp(x)), 0.0)
        o_ref[...] = (jnp.sum(sp) * (1.0 / _B)).reshape(1, 1)

    return pl.pallas_call(
        body,
        out_shape=jax.ShapeDtypeStruct((1, 1), jnp.float32),
    )(logits)


def kernel(centers, contexts, neg_contexts, item_embed, context_embed):
    centers = centers.astype(jnp.int32)
    all_idx = jnp.concatenate(
        [contexts[:, None], neg_contexts], axis=1).astype(jnp.int32)
    all_idx2d = (all_idx * 2).reshape(_B * _KP1 // _GW, _GW)
    item2 = _tc_widen(item_embed.T).reshape(2 * _V, _D)
    ctx2 = _tc_widen(context_embed.T).reshape(2 * _V, _D)
    logits = _sc_logits(centers * 2, all_idx2d, item2, ctx2)
    return _tc_loss(logits.reshape(_B * _KPAD // 128, 128))[0, 0]


# speedup vs baseline: 8.2691x; 1.1838x over previous
import functools

import jax
import jax.numpy as jnp
from jax import lax
from jax.experimental import pallas as pl
from jax.experimental.pallas import tpu as pltpu
from jax.experimental.pallas import tpu_sc as plsc

_D = 64
_B = 16384
_K = 20
_KP1 = _K + 1
_KPAD = 32
_V = 1000000

_NW = 32
_PER_W = _B // _NW
_C = 64
_NCHUNK = _PER_W // _C
_R = _C * _KP1
_GW = 84
_NG = _R // _GW

_TBLK = 8192
_TGRID = -(-_V // _TBLK)

_MESH = dict(core_axis_name="c", subcore_axis_name="s")


def _tc_widen(tab_t):
    def body(x_ref, o_ref):
        xt = jnp.swapaxes(x_ref[...], 0, 1)
        o_ref[...] = jnp.concatenate([xt, xt], axis=1)

    return pl.pallas_call(
        body,
        grid=(_TGRID,),
        in_specs=[pl.BlockSpec((_D, _TBLK), lambda i: (0, i))],
        out_specs=pl.BlockSpec((_TBLK, 128), lambda i: (i, 0)),
        out_shape=jax.ShapeDtypeStruct((_V, 128), jnp.float32),
    )(tab_t)


def _sc_logits(centers, all_idx2d, item2, ctx2):
    mesh = plsc.VectorSubcoreMesh(**_MESH)

    @functools.partial(
        pl.kernel,
        mesh=mesh,
        compiler_params=pltpu.CompilerParams(use_tc_tiling_on_sc=False),
        out_type=jax.ShapeDtypeStruct((_B, _KPAD), jnp.float32),
        scratch_types=[
            pltpu.VMEM((_C,), jnp.int32),
            pltpu.VMEM((_NG, _GW), jnp.int32),
            pltpu.VMEM((_C, _D), jnp.float32),
            pltpu.VMEM((_R, _D), jnp.float32),
            pltpu.VMEM((_C, _KPAD), jnp.float32),
            pltpu.SemaphoreType.DMA,
        ],
    )
    def sc_g(cen_hbm, aidx_hbm, item2_hbm, ctx2_hbm, out_hbm,
             cen_idx_v, aidx_v, cen_rows_v, rows_v, logit_v, sem):
        wid = lax.axis_index("s") * 2 + lax.axis_index("c")
        base_b = wid * _PER_W

        lane = lax.iota(jnp.int32, 16)
        lane_eq = [lane == i for i in range(16)]
        perms = [lane ^ sh for sh in (8, 4, 2, 1)]
        dnums = lax.GatherDimensionNumbers(
            offset_dims=(), collapsed_slice_dims=(0,), start_index_map=(0,))

        def lane_sum(v):
            for perm in perms:
                v = v + lax.gather(
                    v, perm[:, None], dimension_numbers=dnums,
                    slice_sizes=(1,),
                    mode=lax.GatherScatterMode.PROMISE_IN_BOUNDS)
            return v

        for g in range(_NCHUNK):
            b0 = pl.multiple_of(base_b + g * _C, _C)
            pltpu.sync_copy(cen_hbm.at[pl.ds(b0, _C)], cen_idx_v)
            pltpu.sync_copy(
                aidx_hbm.at[pl.ds(pl.multiple_of(b0 * _KP1 // _GW, _NG), _NG),
                            :], aidx_v)
            cps = [pltpu.async_copy(item2_hbm.at[cen_idx_v],
                                    cen_rows_v, sem)]
            for j in range(_NG):
                cps.append(pltpu.async_copy(
                    ctx2_hbm.at[aidx_v.at[j]],
                    rows_v.at[pl.ds(j * _GW, _GW), :], sem))
            for cp in cps:
                cp.wait()

            def body(b, carry):
                c0 = cen_rows_v[b, pl.ds(0, 16)]
                c1 = cen_rows_v[b, pl.ds(16, 16)]
                c2 = cen_rows_v[b, pl.ds(32, 16)]
                c3 = cen_rows_v[b, pl.ds(48, 16)]
                rbase = b * _KP1
                acc0 = jnp.zeros((16,), jnp.float32)
                acc1 = jnp.zeros((16,), jnp.float32)
                for kk in range(_KP1):
                    r = rbase + kk
                    p = (rows_v[r, pl.ds(0, 16)] * c0
                         + rows_v[r, pl.ds(16, 16)] * c1
                         + rows_v[r, pl.ds(32, 16)] * c2
                         + rows_v[r, pl.ds(48, 16)] * c3)
                    s = lane_sum(p)
                    if kk == 0:
                        acc0 = jnp.where(lane_eq[0], -s, acc0)
                    elif kk < 16:
                        acc0 = jnp.where(lane_eq[kk], s, acc0)
                    else:
                        acc1 = jnp.where(lane_eq[kk - 16], s, acc1)
                logit_v[b, pl.ds(0, 16)] = acc0
                logit_v[b, pl.ds(16, 16)] = acc1
                return carry

            lax.fori_loop(0, _C, body, None)
            pltpu.sync_copy(logit_v, out_hbm.at[pl.ds(b0, _C), :])

    return sc_g(centers, all_idx2d, item2, ctx2)


def _tc_loss(logits):
    def body(x_ref, o_ref):
        x = x_ref[...]
        col = lax.broadcasted_iota(jnp.int32, x.shape, 1) % _KPAD
        sp = jnp.where(col < _KP1, jnp.log1p(jnp.exp(x)), 0.0)
        o_ref[...] = (jnp.sum(sp) * (1.0 / _B)).reshape(1, 1)

    return pl.pallas_call(
        body,
        out_shape=jax.ShapeDtypeStruct((1, 1), jnp.float32),
    )(logits)


def kernel(centers, contexts, neg_contexts, item_embed, context_embed):
    centers = centers.astype(jnp.int32)
    all_idx = jnp.concatenate(
        [contexts[:, None], neg_contexts], axis=1).astype(jnp.int32)
    all_idx2d = (all_idx * 2).reshape(_B * _KP1 // _GW, _GW)
    item2 = _tc_widen(item_embed.T).reshape(2 * _V, _D)
    ctx2 = _tc_widen(context_embed.T).reshape(2 * _V, _D)
    logits = _sc_logits(centers * 2, all_idx2d, item2, ctx2)
    return _tc_loss(logits.reshape(_B * _KPAD // 128, 128))[0, 0]


# speedup vs baseline: 9.0787x; 1.0979x over previous
import functools

import jax
import jax.numpy as jnp
from jax import lax
from jax.experimental import pallas as pl
from jax.experimental.pallas import tpu as pltpu
from jax.experimental.pallas import tpu_sc as plsc

_D = 64
_B = 16384
_K = 20
_KP1 = _K + 1
_KPAD = 32
_V = 1000000

_NW = 32
_PER_W = _B // _NW
_C = 64
_NCHUNK = _PER_W // _C
_R = _C * _KP1
_GW = 84
_NG = _R // _GW

_TBLK = 16384
_TGRID = -(-_V // _TBLK)

_MESH = dict(core_axis_name="c", subcore_axis_name="s")


def _tc_widen(tab_t):
    def body(x_ref, o_ref):
        xt = jnp.swapaxes(x_ref[...], 0, 1)
        o_ref[...] = jnp.concatenate([xt, xt], axis=1)

    return pl.pallas_call(
        body,
        grid=(_TGRID,),
        in_specs=[pl.BlockSpec((_D, _TBLK), lambda i: (0, i))],
        out_specs=pl.BlockSpec((_TBLK, 128), lambda i: (i, 0)),
        out_shape=jax.ShapeDtypeStruct((_V, 128), jnp.float32),
    )(tab_t)


def _sc_logits(centers, all_idx2d, item2, ctx2):
    mesh = plsc.VectorSubcoreMesh(**_MESH)

    @functools.partial(
        pl.kernel,
        mesh=mesh,
        compiler_params=pltpu.CompilerParams(use_tc_tiling_on_sc=False),
        out_type=jax.ShapeDtypeStruct((_B, _KPAD), jnp.float32),
        scratch_types=[
            pltpu.VMEM((_C,), jnp.int32),
            pltpu.VMEM((_NG, _GW), jnp.int32),
            pltpu.VMEM((_C, _D), jnp.float32),
            pltpu.VMEM((_R, _D), jnp.float32),
            pltpu.VMEM((_C, _KPAD), jnp.float32),
            pltpu.SemaphoreType.DMA,
        ],
    )
    def sc_g(cen_hbm, aidx_hbm, item2_hbm, ctx2_hbm, out_hbm,
             cen_idx_v, aidx_v, cen_rows_v, rows_v, logit_v, sem):
        wid = lax.axis_index("s") * 2 + lax.axis_index("c")
        base_b = wid * _PER_W

        lane = lax.iota(jnp.int32, 16)
        lane_eq = [lane == i for i in range(16)]
        perms = [lane ^ sh for sh in (8, 4, 2, 1)]
        dnums = lax.GatherDimensionNumbers(
            offset_dims=(), collapsed_slice_dims=(0,), start_index_map=(0,))

        def lane_sum(v):
            for perm in perms:
                v = v + lax.gather(
                    v, perm[:, None], dimension_numbers=dnums,
                    slice_sizes=(1,),
                    mode=lax.GatherScatterMode.PROMISE_IN_BOUNDS)
            return v

        for g in range(_NCHUNK):
            b0 = pl.multiple_of(base_b + g * _C, _C)
            pltpu.sync_copy(cen_hbm.at[pl.ds(b0, _C)], cen_idx_v)
            pltpu.sync_copy(
                aidx_hbm.at[pl.ds(pl.multiple_of(b0 * _KP1 // _GW, _NG), _NG),
                            :], aidx_v)
            cps = [pltpu.async_copy(item2_hbm.at[cen_idx_v],
                                    cen_rows_v, sem)]
            for j in range(_NG):
                cps.append(pltpu.async_copy(
                    ctx2_hbm.at[aidx_v.at[j]],
                    rows_v.at[pl.ds(j * _GW, _GW), :], sem))
            for cp in cps:
                cp.wait()

            def body(b, carry):
                c0 = cen_rows_v[b, pl.ds(0, 16)]
                c1 = cen_rows_v[b, pl.ds(16, 16)]
                c2 = cen_rows_v[b, pl.ds(32, 16)]
                c3 = cen_rows_v[b, pl.ds(48, 16)]
                rbase = b * _KP1
                acc0 = jnp.zeros((16,), jnp.float32)
                acc1 = jnp.zeros((16,), jnp.float32)
                for kk in range(_KP1):
                    r = rbase + kk
                    p = (rows_v[r, pl.ds(0, 16)] * c0
                         + rows_v[r, pl.ds(16, 16)] * c1
                         + rows_v[r, pl.ds(32, 16)] * c2
                         + rows_v[r, pl.ds(48, 16)] * c3)
                    s = lane_sum(p)
                    if kk == 0:
                        acc0 = jnp.where(lane_eq[0], -s, acc0)
                    elif kk < 16:
                        acc0 = jnp.where(lane_eq[kk], s, acc0)
                    else:
                        acc1 = jnp.where(lane_eq[kk - 16], s, acc1)
                logit_v[b, pl.ds(0, 16)] = acc0
                logit_v[b, pl.ds(16, 16)] = acc1
                return carry

            lax.fori_loop(0, _C, body, None)
            pltpu.sync_copy(logit_v, out_hbm.at[pl.ds(b0, _C), :])

    return sc_g(centers, all_idx2d, item2, ctx2)


def _tc_loss(logits):
    def body(x_ref, o_ref):
        x = x_ref[...]
        col = lax.broadcasted_iota(jnp.int32, x.shape, 1) % _KPAD
        sp = jnp.where(col < _KP1, jnp.log1p(jnp.exp(x)), 0.0)
        o_ref[...] = (jnp.sum(sp) * (1.0 / _B)).reshape(1, 1)

    return pl.pallas_call(
        body,
        out_shape=jax.ShapeDtypeStruct((1, 1), jnp.float32),
    )(logits)


def kernel(centers, contexts, neg_contexts, item_embed, context_embed):
    centers = centers.astype(jnp.int32)
    all_idx = jnp.concatenate(
        [contexts[:, None], neg_contexts], axis=1).astype(jnp.int32)
    all_idx2d = (all_idx * 2).reshape(_B * _KP1 // _GW, _GW)
    item2 = _tc_widen(item_embed.T).reshape(2 * _V, _D)
    ctx2 = _tc_widen(context_embed.T).reshape(2 * _V, _D)
    logits = _sc_logits(centers * 2, all_idx2d, item2, ctx2)
    return _tc_loss(logits.reshape(_B * _KPAD // 128, 128))[0, 0]


# speedup vs baseline: 11.5764x; 1.2751x over previous
import functools

import jax
import jax.numpy as jnp
from jax import lax
from jax.experimental import pallas as pl
from jax.experimental.pallas import tpu as pltpu
from jax.experimental.pallas import tpu_sc as plsc

_D = 64
_B = 16384
_K = 20
_KP1 = _K + 1
_KPAD = 32
_V = 1000000

_NW = 32
_PER_W = _B // _NW
_C = 64
_NCHUNK = _PER_W // _C
_R = _C * _KP1
_GW = 84
_NG = _R // _GW

_TBLK = 16384
_TGRID = -(-_V // (2 * _TBLK))
_VROWS = _TGRID * _TBLK

_MESH = dict(core_axis_name="c", subcore_axis_name="s")


def _tc_widen(tab_t):
    def body(x0_ref, x1_ref, o_ref):
        o_ref[...] = jnp.concatenate(
            [jnp.swapaxes(x0_ref[...], 0, 1),
             jnp.swapaxes(x1_ref[...], 0, 1)], axis=1)

    return pl.pallas_call(
        body,
        grid=(_TGRID,),
        in_specs=[
            pl.BlockSpec((_D, _TBLK), lambda i: (0, 2 * i)),
            pl.BlockSpec((_D, _TBLK), lambda i: (0, 2 * i + 1)),
        ],
        out_specs=pl.BlockSpec((_TBLK, 128), lambda i: (i, 0)),
        out_shape=jax.ShapeDtypeStruct((_VROWS, 128), jnp.float32),
    )(tab_t, tab_t)


def _sc_logits(centers, all_idx2d, item2, ctx2):
    mesh = plsc.VectorSubcoreMesh(**_MESH)

    @functools.partial(
        pl.kernel,
        mesh=mesh,
        compiler_params=pltpu.CompilerParams(use_tc_tiling_on_sc=False),
        out_type=jax.ShapeDtypeStruct((_B, _KPAD), jnp.float32),
        scratch_types=[
            pltpu.VMEM((_C,), jnp.int32),
            pltpu.VMEM((_NG, _GW), jnp.int32),
            pltpu.VMEM((_C, _D), jnp.float32),
            pltpu.VMEM((_R, _D), jnp.float32),
            pltpu.VMEM((_C, _KPAD), jnp.float32),
            pltpu.SemaphoreType.DMA,
        ],
    )
    def sc_g(cen_hbm, aidx_hbm, item2_hbm, ctx2_hbm, out_hbm,
             cen_idx_v, aidx_v, cen_rows_v, rows_v, logit_v, sem):
        wid = lax.axis_index("s") * 2 + lax.axis_index("c")
        base_b = wid * _PER_W

        lane = lax.iota(jnp.int32, 16)
        lane_eq = [lane == i for i in range(16)]
        perms = [lane ^ sh for sh in (8, 4, 2, 1)]
        dnums = lax.GatherDimensionNumbers(
            offset_dims=(), collapsed_slice_dims=(0,), start_index_map=(0,))

        def lane_sum(v):
            for perm in perms:
                v = v + lax.gather(
                    v, perm[:, None], dimension_numbers=dnums,
                    slice_sizes=(1,),
                    mode=lax.GatherScatterMode.PROMISE_IN_BOUNDS)
            return v

        for g in range(_NCHUNK):
            b0 = pl.multiple_of(base_b + g * _C, _C)
            pltpu.sync_copy(cen_hbm.at[pl.ds(b0, _C)], cen_idx_v)
            pltpu.sync_copy(
                aidx_hbm.at[pl.ds(pl.multiple_of(b0 * _KP1 // _GW, _NG), _NG),
                            :], aidx_v)
            cps = [pltpu.async_copy(item2_hbm.at[cen_idx_v],
                                    cen_rows_v, sem)]
            for j in range(_NG):
                cps.append(pltpu.async_copy(
                    ctx2_hbm.at[aidx_v.at[j]],
                    rows_v.at[pl.ds(j * _GW, _GW), :], sem))
            for cp in cps:
                cp.wait()

            def body(b, carry):
                c0 = cen_rows_v[b, pl.ds(0, 16)]
                c1 = cen_rows_v[b, pl.ds(16, 16)]
                c2 = cen_rows_v[b, pl.ds(32, 16)]
                c3 = cen_rows_v[b, pl.ds(48, 16)]
                rbase = b * _KP1
                acc0 = jnp.zeros((16,), jnp.float32)
                acc1 = jnp.zeros((16,), jnp.float32)
                for kk in range(_KP1):
                    r = rbase + kk
                    p = (rows_v[r, pl.ds(0, 16)] * c0
                         + rows_v[r, pl.ds(16, 16)] * c1
                         + rows_v[r, pl.ds(32, 16)] * c2
                         + rows_v[r, pl.ds(48, 16)] * c3)
                    s = lane_sum(p)
                    if kk == 0:
                        acc0 = jnp.where(lane_eq[0], -s, acc0)
                    elif kk < 16:
                        acc0 = jnp.where(lane_eq[kk], s, acc0)
                    else:
                        acc1 = jnp.where(lane_eq[kk - 16], s, acc1)
                logit_v[b, pl.ds(0, 16)] = acc0
                logit_v[b, pl.ds(16, 16)] = acc1
                return carry

            lax.fori_loop(0, _C, body, None)
            pltpu.sync_copy(logit_v, out_hbm.at[pl.ds(b0, _C), :])

    return sc_g(centers, all_idx2d, item2, ctx2)


def _tc_loss(logits):
    def body(x_ref, o_ref):
        x = x_ref[...]
        col = lax.broadcasted_iota(jnp.int32, x.shape, 1) % _KPAD
        sp = jnp.where(col < _KP1, jnp.log1p(jnp.exp(x)), 0.0)
        o_ref[...] = (jnp.sum(sp) * (1.0 / _B)).reshape(1, 1)

    return pl.pallas_call(
        body,
        out_shape=jax.ShapeDtypeStruct((1, 1), jnp.float32),
    )(logits)


def kernel(centers, contexts, neg_contexts, item_embed, context_embed):
    centers = centers.astype(jnp.int32)
    all_idx = jnp.concatenate(
        [contexts[:, None], neg_contexts], axis=1).astype(jnp.int32)
    def lin_idx(v):
        q = v // (2 * _TBLK)
        half = (v // _TBLK) & 1
        m = v & (_TBLK - 1)
        return 2 * (q * _TBLK + m) + half

    all_idx2d = lin_idx(all_idx).reshape(_B * _KP1 // _GW, _GW)
    item2 = _tc_widen(item_embed.T).reshape(2 * _VROWS, _D)
    ctx2 = _tc_widen(context_embed.T).reshape(2 * _VROWS, _D)
    logits = _sc_logits(lin_idx(centers), all_idx2d, item2, ctx2)
    return _tc_loss(logits.reshape(_B * _KPAD // 128, 128))[0, 0]


# speedup vs baseline: 12.0598x; 1.0418x over previous
import functools

import jax
import jax.numpy as jnp
from jax import lax
from jax.experimental import pallas as pl
from jax.experimental.pallas import tpu as pltpu
from jax.experimental.pallas import tpu_sc as plsc

_D = 64
_B = 16384
_K = 20
_KP1 = _K + 1
_KPAD = 32
_V = 1000000

_NW = 32
_PER_W = _B // _NW
_C = 32
_NCHUNK = _PER_W // _C
_R = _C * _KP1
_GW = 84
_NG = _R // _GW

_TBLK = 16384
_TGRID = -(-_V // (2 * _TBLK))
_VROWS = _TGRID * _TBLK

_MESH = dict(core_axis_name="c", subcore_axis_name="s")


def _tc_widen(tab_t):
    def body(x0_ref, x1_ref, o_ref):
        o_ref[...] = jnp.concatenate(
            [jnp.swapaxes(x0_ref[...], 0, 1),
             jnp.swapaxes(x1_ref[...], 0, 1)], axis=1)

    return pl.pallas_call(
        body,
        grid=(_TGRID,),
        in_specs=[
            pl.BlockSpec((_D, _TBLK), lambda i: (0, 2 * i)),
            pl.BlockSpec((_D, _TBLK), lambda i: (0, 2 * i + 1)),
        ],
        out_specs=pl.BlockSpec((_TBLK, 128), lambda i: (i, 0)),
        out_shape=jax.ShapeDtypeStruct((_VROWS, 128), jnp.float32),
    )(tab_t, tab_t)


def _sc_logits(centers, all_idx2d, item2, ctx2):
    mesh = plsc.VectorSubcoreMesh(**_MESH)

    @functools.partial(
        pl.kernel,
        mesh=mesh,
        compiler_params=pltpu.CompilerParams(use_tc_tiling_on_sc=False),
        out_type=jax.ShapeDtypeStruct((_B, _KPAD), jnp.float32),
        scratch_types=[
            pltpu.VMEM((_C,), jnp.int32),
            pltpu.VMEM((_C,), jnp.int32),
            pltpu.VMEM((_NG, _GW), jnp.int32),
            pltpu.VMEM((_NG, _GW), jnp.int32),
            pltpu.VMEM((_C, _D), jnp.float32),
            pltpu.VMEM((_C, _D), jnp.float32),
            pltpu.VMEM((_R, _D), jnp.float32),
            pltpu.VMEM((_R, _D), jnp.float32),
            pltpu.VMEM((_C, _KPAD), jnp.float32),
            pltpu.SemaphoreType.DMA,
            pltpu.SemaphoreType.DMA,
        ],
    )
    def sc_g(cen_hbm, aidx_hbm, item2_hbm, ctx2_hbm, out_hbm,
             cen_idx_v0, cen_idx_v1, aidx_v0, aidx_v1, cen_rows_v0,
             cen_rows_v1, rows_v0, rows_v1, logit_v, sem0, sem1):
        wid = lax.axis_index("s") * 2 + lax.axis_index("c")
        base_b = wid * _PER_W
        bufs = [
            (cen_idx_v0, aidx_v0, cen_rows_v0, rows_v0, sem0),
            (cen_idx_v1, aidx_v1, cen_rows_v1, rows_v1, sem1),
        ]

        lane = lax.iota(jnp.int32, 16)
        lane_eq = [lane == i for i in range(16)]
        perms = [lane ^ sh for sh in (8, 4, 2, 1)]
        dnums = lax.GatherDimensionNumbers(
            offset_dims=(), collapsed_slice_dims=(0,), start_index_map=(0,))

        def lane_sum(v):
            for perm in perms:
                v = v + lax.gather(
                    v, perm[:, None], dimension_numbers=dnums,
                    slice_sizes=(1,),
                    mode=lax.GatherScatterMode.PROMISE_IN_BOUNDS)
            return v

        def start(g, buf):
            cen_idx_v, aidx_v, cen_rows_v, rows_v, sem = buf
            b0 = pl.multiple_of(base_b + g * _C, _C)
            pltpu.sync_copy(cen_hbm.at[pl.ds(b0, _C)], cen_idx_v)
            pltpu.sync_copy(
                aidx_hbm.at[pl.ds(pl.multiple_of(b0 * _KP1 // _GW, _NG), _NG),
                            :], aidx_v)
            cps = [pltpu.async_copy(item2_hbm.at[cen_idx_v],
                                    cen_rows_v, sem)]
            for j in range(_NG):
                cps.append(pltpu.async_copy(
                    ctx2_hbm.at[aidx_v.at[j]],
                    rows_v.at[pl.ds(j * _GW, _GW), :], sem))
            return cps

        pend = start(0, bufs[0])
        for g in range(_NCHUNK):
            cen_idx_v, aidx_v, cen_rows_v, rows_v, sem = bufs[g % 2]
            b0 = pl.multiple_of(base_b + g * _C, _C)
            nxt = (start(g + 1, bufs[(g + 1) % 2])
                   if g + 1 < _NCHUNK else [])
            for cp in pend:
                cp.wait()
            pend = nxt

            def body(b, carry):
                c0 = cen_rows_v[b, pl.ds(0, 16)]
                c1 = cen_rows_v[b, pl.ds(16, 16)]
                c2 = cen_rows_v[b, pl.ds(32, 16)]
                c3 = cen_rows_v[b, pl.ds(48, 16)]
                rbase = b * _KP1
                acc0 = jnp.zeros((16,), jnp.float32)
                acc1 = jnp.zeros((16,), jnp.float32)
                for kk in range(_KP1):
                    r = rbase + kk
                    p = (rows_v[r, pl.ds(0, 16)] * c0
                         + rows_v[r, pl.ds(16, 16)] * c1
                         + rows_v[r, pl.ds(32, 16)] * c2
                         + rows_v[r, pl.ds(48, 16)] * c3)
                    s = lane_sum(p)
                    if kk == 0:
                        acc0 = jnp.where(lane_eq[0], -s, acc0)
                    elif kk < 16:
                        acc0 = jnp.where(lane_eq[kk], s, acc0)
                    else:
                        acc1 = jnp.where(lane_eq[kk - 16], s, acc1)
                logit_v[b, pl.ds(0, 16)] = acc0
                logit_v[b, pl.ds(16, 16)] = acc1
                return carry

            lax.fori_loop(0, _C, body, None)
            pltpu.sync_copy(logit_v, out_hbm.at[pl.ds(b0, _C), :])

    return sc_g(centers, all_idx2d, item2, ctx2)


def _tc_loss(logits):
    def body(x_ref, o_ref):
        x = x_ref[...]
        col = lax.broadcasted_iota(jnp.int32, x.shape, 1) % _KPAD
        sp = jnp.where(col < _KP1, jnp.log1p(jnp.exp(x)), 0.0)
        o_ref[...] = (jnp.sum(sp) * (1.0 / _B)).reshape(1, 1)

    return pl.pallas_call(
        body,
        out_shape=jax.ShapeDtypeStruct((1, 1), jnp.float32),
    )(logits)


def kernel(centers, contexts, neg_contexts, item_embed, context_embed):
    centers = centers.astype(jnp.int32)
    all_idx = jnp.concatenate(
        [contexts[:, None], neg_contexts], axis=1).astype(jnp.int32)
    def lin_idx(v):
        q = v // (2 * _TBLK)
        half = (v // _TBLK) & 1
        m = v & (_TBLK - 1)
        return 2 * (q * _TBLK + m) + half

    all_idx2d = lin_idx(all_idx).reshape(_B * _KP1 // _GW, _GW)
    item2 = _tc_widen(item_embed.T).reshape(2 * _VROWS, _D)
    ctx2 = _tc_widen(context_embed.T).reshape(2 * _VROWS, _D)
    logits = _sc_logits(lin_idx(centers), all_idx2d, item2, ctx2)
    return _tc_loss(logits.reshape(_B * _KPAD // 128, 128))[0, 0]
